# Initial kernel scaffold; baseline (speedup 1.0000x reference)
import jax
import jax.numpy as jnp
from jax.experimental import pallas as pl


def kernel(x, edge_index, W1, b1, W2, b2, W3, b3):
    raise NotImplementedError("write your pallas kernel here")


# speedup vs baseline: 3.0336x; 3.0336x over previous
import functools

import jax
import jax.numpy as jnp
from jax import lax
from jax.experimental import pallas as pl
from jax.experimental.pallas import tpu as pltpu
from jax.experimental.pallas import tpu_sc as plsc

N = 10000
E = 320000
D = 128
H = 128
C = 47

NC = 2
NS = 16
NW = NC * NS
CHUNK = 128
NCHUNK = 80
EPW = CHUNK * NCHUNK
EPAD = EPW * NW
NACC = 10112
RPS = NACC // NS


def _propagate_body(z, srcp, dstp, zzero, out, acc, srcv, dstv, rows, sem):
    c = lax.axis_index("c")
    s = lax.axis_index("s")
    wid = c * NS + s
    pltpu.sync_copy(zzero.at[pl.ds(s * RPS, RPS)], acc.at[pl.ds(s * RPS, RPS)])
    pltpu.sync_copy(srcp.at[wid], srcv)
    pltpu.sync_copy(dstp.at[wid], dstv)
    plsc.subcore_barrier()

    def step(j, carry):
        pltpu.async_copy(z.at[srcv.at[j]], rows, sem).wait()
        pltpu.sync_copy(rows, acc.at[dstv.at[j]], add=True)
        return carry

    lax.fori_loop(0, NCHUNK, step, 0)
    plsc.subcore_barrier()
    pltpu.sync_copy(acc.at[pl.ds(s * RPS, RPS)], out.at[c, pl.ds(s * RPS, RPS)])


@functools.partial(
    pl.kernel,
    out_type=jax.ShapeDtypeStruct((NC, NACC, D), jnp.float32),
    mesh=plsc.VectorSubcoreMesh(core_axis_name="c", subcore_axis_name="s"),
    scratch_types=[
        pltpu.VMEM_SHARED((NACC, D), jnp.float32),
        pltpu.VMEM((NCHUNK, CHUNK), jnp.int32),
        pltpu.VMEM((NCHUNK, CHUNK), jnp.int32),
        pltpu.VMEM((CHUNK, D), jnp.float32),
        pltpu.SemaphoreType.DMA,
    ],
)
def _propagate(z, srcp, dstp, zzero, out, acc, srcv, dstv, rows, sem):
    _propagate_body(z, srcp, dstp, zzero, out, acc, srcv, dstv, rows, sem)


def _mm_first_body(x_ref, w_ref, o_ref):
    o_ref[...] = lax.dot_general(
        x_ref[...], w_ref[...], (((1,), (1,)), ((), ())),
        preferred_element_type=jnp.float32)


def _fuse_body(p_ref, b_ref, w_ref, o_ref):
    h = p_ref[0, :N, :] + p_ref[1, :N, :] + b_ref[...]
    h = jnp.maximum(h, 0.0)
    o_ref[...] = lax.dot_general(
        h, w_ref[...], (((1,), (1,)), ((), ())),
        preferred_element_type=jnp.float32)


def _final_body(p_ref, b_ref, o_ref):
    v = p_ref[0, :N, :] + p_ref[1, :N, :] + b_ref[...]
    col = lax.broadcasted_iota(jnp.int32, (N, D), 1)
    valid = col < C
    vm = jnp.where(valid, v, -jnp.inf)
    m = jnp.max(vm, axis=1, keepdims=True)
    ex = jnp.where(valid, jnp.exp(v - m), 0.0)
    lse = jnp.log(jnp.sum(ex, axis=1, keepdims=True)) + m
    o_ref[...] = v - lse


_mm_first = pl.pallas_call(
    _mm_first_body,
    out_shape=jax.ShapeDtypeStruct((N, H), jnp.float32),
)

_fuse = pl.pallas_call(
    _fuse_body,
    out_shape=jax.ShapeDtypeStruct((N, H), jnp.float32),
)

_final = pl.pallas_call(
    _final_body,
    out_shape=jax.ShapeDtypeStruct((N, D), jnp.float32),
)


def kernel(x, edge_index, W1, b1, W2, b2, W3, b3):
    src = edge_index[0].astype(jnp.int32)
    dst = edge_index[1].astype(jnp.int32)
    pad = EPAD - E
    srcp = jnp.concatenate([src, jnp.zeros((pad,), jnp.int32)]).reshape(NW, NCHUNK, CHUNK)
    dstp = jnp.concatenate([dst, jnp.full((pad,), N, jnp.int32)]).reshape(NW, NCHUNK, CHUNK)
    zzero = jnp.zeros((NACC, D), jnp.float32)

    W3p = jnp.zeros((D, H), jnp.float32).at[:C, :].set(W3)
    b3p = jnp.zeros((1, D), jnp.float32).at[0, :C].set(b3)

    z1 = _mm_first(x, W1)
    p1 = _propagate(z1, srcp, dstp, zzero)
    z2 = _fuse(p1, b1.reshape(1, H), W2)
    p2 = _propagate(z2, srcp, dstp, zzero)
    z3 = _fuse(p2, b2.reshape(1, H), W3p)
    p3 = _propagate(z3, srcp, dstp, zzero)
    o = _final(p3, b3p)
    return o[:, :C]


# speedup vs baseline: 4.1526x; 1.3689x over previous
import functools

import jax
import jax.numpy as jnp
from jax import lax
from jax.experimental import pallas as pl
from jax.experimental.pallas import tpu as pltpu
from jax.experimental.pallas import tpu_sc as plsc

N = 10000
E = 320000
D = 128
H = 128
C = 47

NC = 2
NS = 16
HD = D // 2
CHUNK = 128
NCHUNK = 160
EPW = CHUNK * NCHUNK
EPAD = EPW * NS
NACC = 10112
RPS = NACC // NS


def _propagate_body(z, srcp, dstp, zzero, out, acc, srcv, dstv, rows0, rows1, sem):
    c = lax.axis_index("c")
    s = lax.axis_index("s")
    pltpu.sync_copy(zzero.at[pl.ds(s * RPS, RPS)], acc.at[pl.ds(s * RPS, RPS)])
    pltpu.sync_copy(srcp.at[s], srcv)
    pltpu.sync_copy(dstp.at[s], dstv)
    plsc.subcore_barrier()

    zc = z.at[c]

    pltpu.async_copy(zc.at[srcv.at[0]], rows0, sem)

    def step(j2, carry):
        a = 2 * j2
        b = a + 1
        pltpu.make_async_copy(zc.at[srcv.at[a]], rows0, sem).wait()
        pltpu.async_copy(zc.at[srcv.at[b]], rows1, sem)
        pltpu.sync_copy(rows0, acc.at[dstv.at[a]], add=True)
        pltpu.make_async_copy(zc.at[srcv.at[b]], rows1, sem).wait()
        nxt = jnp.minimum(b + 1, NCHUNK - 1)
        pltpu.async_copy(zc.at[srcv.at[nxt]], rows0, sem)
        pltpu.sync_copy(rows1, acc.at[dstv.at[b]], add=True)
        return carry

    lax.fori_loop(0, NCHUNK // 2, step, 0)
    pltpu.make_async_copy(zc.at[srcv.at[0]], rows0, sem).wait()
    plsc.subcore_barrier()
    pltpu.sync_copy(acc.at[pl.ds(s * RPS, RPS)], out.at[c, pl.ds(s * RPS, RPS)])


@functools.partial(
    pl.kernel,
    out_type=jax.ShapeDtypeStruct((NC, NACC, HD), jnp.float32),
    mesh=plsc.VectorSubcoreMesh(core_axis_name="c", subcore_axis_name="s"),
    scratch_types=[
        pltpu.VMEM_SHARED((NACC, HD), jnp.float32),
        pltpu.VMEM((NCHUNK, CHUNK), jnp.int32),
        pltpu.VMEM((NCHUNK, CHUNK), jnp.int32),
        pltpu.VMEM((CHUNK, HD), jnp.float32),
        pltpu.VMEM((CHUNK, HD), jnp.float32),
        pltpu.SemaphoreType.DMA,
    ],
    compiler_params=pltpu.CompilerParams(use_tc_tiling_on_sc=False),
)
def _propagate(z, srcp, dstp, zzero, out, acc, srcv, dstv, rows0, rows1, sem):
    _propagate_body(z, srcp, dstp, zzero, out, acc, srcv, dstv, rows0, rows1, sem)


def _split_cols(o_ref, res):
    o_ref[0, :, :] = res[:, :HD]
    o_ref[1, :, :] = res[:, HD:]


def _mm_first_body(x_ref, w_ref, o_ref):
    res = lax.dot_general(
        x_ref[...], w_ref[...], (((1,), (1,)), ((), ())),
        preferred_element_type=jnp.float32)
    _split_cols(o_ref, res)


def _fuse_body(p_ref, b_ref, w_ref, o_ref):
    h = jnp.concatenate([p_ref[0, :N, :], p_ref[1, :N, :]], axis=1) + b_ref[...]
    h = jnp.maximum(h, 0.0)
    res = lax.dot_general(
        h, w_ref[...], (((1,), (1,)), ((), ())),
        preferred_element_type=jnp.float32)
    _split_cols(o_ref, res)


def _final_body(p_ref, b_ref, o_ref):
    v = jnp.concatenate([p_ref[0, :N, :], p_ref[1, :N, :]], axis=1) + b_ref[...]
    col = lax.broadcasted_iota(jnp.int32, (N, D), 1)
    valid = col < C
    vm = jnp.where(valid, v, -jnp.inf)
    m = jnp.max(vm, axis=1, keepdims=True)
    ex = jnp.where(valid, jnp.exp(v - m), 0.0)
    lse = jnp.log(jnp.sum(ex, axis=1, keepdims=True)) + m
    o_ref[...] = v - lse


_mm_first = pl.pallas_call(
    _mm_first_body,
    out_shape=jax.ShapeDtypeStruct((NC, N, HD), jnp.float32),
)

_fuse = pl.pallas_call(
    _fuse_body,
    out_shape=jax.ShapeDtypeStruct((NC, N, HD), jnp.float32),
)

_final = pl.pallas_call(
    _final_body,
    out_shape=jax.ShapeDtypeStruct((N, D), jnp.float32),
)


def kernel(x, edge_index, W1, b1, W2, b2, W3, b3):
    src = edge_index[0].astype(jnp.int32)
    dst = edge_index[1].astype(jnp.int32)
    pad = EPAD - E
    srcp = jnp.concatenate([src, jnp.zeros((pad,), jnp.int32)]).reshape(NS, NCHUNK, CHUNK)
    dstp = jnp.concatenate([dst, jnp.full((pad,), N, jnp.int32)]).reshape(NS, NCHUNK, CHUNK)
    zzero = jnp.zeros((NACC, HD), jnp.float32)

    W3p = jnp.zeros((D, H), jnp.float32).at[:C, :].set(W3)
    b3p = jnp.zeros((1, D), jnp.float32).at[0, :C].set(b3)

    z1 = _mm_first(x, W1)
    p1 = _propagate(z1, srcp, dstp, zzero)
    z2 = _fuse(p1, b1.reshape(1, H), W2)
    p2 = _propagate(z2, srcp, dstp, zzero)
    z3 = _fuse(p2, b2.reshape(1, H), W3p)
    p3 = _propagate(z3, srcp, dstp, zzero)
    o = _final(p3, b3p)
    return o[:, :C]


# speedup vs baseline: 4.6981x; 1.1314x over previous
import functools

import jax
import jax.numpy as jnp
from jax import lax
from jax.experimental import pallas as pl
from jax.experimental.pallas import tpu as pltpu
from jax.experimental.pallas import tpu_sc as plsc

N = 10000
E = 320000
D = 128
H = 128
C = 47

NC = 2
NS = 16
HD = D // 2
CHUNK = 128
NCHUNK = 160
EPW = CHUNK * NCHUNK
EPAD = EPW * NS
NACC = 10112
RPS = NACC // NS


HCHUNK = NCHUNK // 2


def _propagate_body(z, srcp, dstp, zzero, out, acc, srcv, dstv,
                    rows, gsems, ssems):
    c = lax.axis_index("c")
    s = lax.axis_index("s")
    pltpu.sync_copy(zzero.at[pl.ds(s * RPS, RPS)], acc.at[pl.ds(s * RPS, RPS)])

    zc = z.at[c]

    def issue_g(b, j):
        pltpu.async_copy(zc.at[srcv.at[j]], rows[b], gsems[b])

    def wait_g(b):
        pltpu.make_async_copy(zc.at[srcv.at[0]], rows[b], gsems[b]).wait()

    def issue_s(b, j):
        pltpu.async_copy(rows[b], acc.at[dstv.at[j]], ssems[b], add=True)

    def wait_s(b):
        pltpu.make_async_copy(rows[b], acc.at[dstv.at[0]], ssems[b]).wait()

    for phase in range(2):
        pltpu.sync_copy(srcp.at[s, pl.ds(phase * HCHUNK, HCHUNK)], srcv)
        pltpu.sync_copy(dstp.at[s, pl.ds(phase * HCHUNK, HCHUNK)], dstv)
        if phase == 0:
            plsc.subcore_barrier()
        issue_g(0, 0)
        issue_g(1, 1)
        wait_g(0)
        issue_s(0, 0)
        issue_g(2, 2)
        wait_g(1)
        issue_s(1, 1)
        issue_g(3, 3)

        def steady(j2, carry):
            j0 = 2 + 4 * j2
            for b4 in range(4):
                j = j0 + b4
                b = (2 + b4) % 4
                cc = b4
                wait_g(b)
                issue_s(b, j)
                wait_s(cc)
                issue_g(cc, j + 2)
            return carry

        lax.fori_loop(0, (HCHUNK - 4) // 4, steady, 0)
        wait_g(2)
        issue_s(2, HCHUNK - 2)
        wait_s(0)
        wait_g(3)
        issue_s(3, HCHUNK - 1)
        wait_s(1)
        wait_s(2)
        wait_s(3)

    plsc.subcore_barrier()
    pltpu.sync_copy(acc.at[pl.ds(s * RPS, RPS)], out.at[c, pl.ds(s * RPS, RPS)])


@functools.partial(
    pl.kernel,
    out_type=jax.ShapeDtypeStruct((NC, NACC, HD), jnp.float32),
    mesh=plsc.VectorSubcoreMesh(core_axis_name="c", subcore_axis_name="s"),
    scratch_types=[
        pltpu.VMEM_SHARED((NACC, HD), jnp.float32),
        pltpu.VMEM((HCHUNK, CHUNK), jnp.int32),
        pltpu.VMEM((HCHUNK, CHUNK), jnp.int32),
        [pltpu.VMEM((CHUNK, HD), jnp.float32) for _ in range(4)],
        [pltpu.SemaphoreType.DMA for _ in range(4)],
        [pltpu.SemaphoreType.DMA for _ in range(4)],
    ],
    compiler_params=pltpu.CompilerParams(use_tc_tiling_on_sc=False),
)
def _propagate(z, srcp, dstp, zzero, out, acc, srcv, dstv, rows, gsems, ssems):
    _propagate_body(z, srcp, dstp, zzero, out, acc, srcv, dstv, rows, gsems, ssems)


def _split_cols(o_ref, res):
    o_ref[0, :, :] = res[:, :HD]
    o_ref[1, :, :] = res[:, HD:]


def _mm_first_body(x_ref, w_ref, o_ref):
    res = lax.dot_general(
        x_ref[...], w_ref[...], (((1,), (1,)), ((), ())),
        preferred_element_type=jnp.float32)
    _split_cols(o_ref, res)


def _fuse_body(p_ref, b_ref, w_ref, o_ref):
    h = jnp.concatenate([p_ref[0, :N, :], p_ref[1, :N, :]], axis=1) + b_ref[...]
    h = jnp.maximum(h, 0.0)
    res = lax.dot_general(
        h, w_ref[...], (((1,), (1,)), ((), ())),
        preferred_element_type=jnp.float32)
    _split_cols(o_ref, res)


def _final_body(p_ref, b_ref, o_ref):
    v = jnp.concatenate([p_ref[0, :N, :], p_ref[1, :N, :]], axis=1) + b_ref[...]
    col = lax.broadcasted_iota(jnp.int32, (N, D), 1)
    valid = col < C
    vm = jnp.where(valid, v, -jnp.inf)
    m = jnp.max(vm, axis=1, keepdims=True)
    ex = jnp.where(valid, jnp.exp(v - m), 0.0)
    lse = jnp.log(jnp.sum(ex, axis=1, keepdims=True)) + m
    o_ref[...] = v - lse


_mm_first = pl.pallas_call(
    _mm_first_body,
    out_shape=jax.ShapeDtypeStruct((NC, N, HD), jnp.float32),
)

_fuse = pl.pallas_call(
    _fuse_body,
    out_shape=jax.ShapeDtypeStruct((NC, N, HD), jnp.float32),
)

_final = pl.pallas_call(
    _final_body,
    out_shape=jax.ShapeDtypeStruct((N, D), jnp.float32),
)


def kernel(x, edge_index, W1, b1, W2, b2, W3, b3):
    src = edge_index[0].astype(jnp.int32)
    dst = edge_index[1].astype(jnp.int32)
    pad = EPAD - E
    srcp = jnp.concatenate([src, jnp.zeros((pad,), jnp.int32)]).reshape(NS, NCHUNK, CHUNK)
    dstp = jnp.concatenate([dst, jnp.full((pad,), N, jnp.int32)]).reshape(NS, NCHUNK, CHUNK)
    zzero = jnp.zeros((NACC, HD), jnp.float32)

    W3p = jnp.zeros((D, H), jnp.float32).at[:C, :].set(W3)
    b3p = jnp.zeros((1, D), jnp.float32).at[0, :C].set(b3)

    z1 = _mm_first(x, W1)
    p1 = _propagate(z1, srcp, dstp, zzero)
    z2 = _fuse(p1, b1.reshape(1, H), W2)
    p2 = _propagate(z2, srcp, dstp, zzero)
    z3 = _fuse(p2, b2.reshape(1, H), W3p)
    p3 = _propagate(z3, srcp, dstp, zzero)
    o = _final(p3, b3p)
    return o[:, :C]


# speedup vs baseline: 9.3755x; 1.9956x over previous
import functools

import jax
import jax.numpy as jnp
from jax import lax
from jax.experimental import pallas as pl
from jax.experimental.pallas import tpu as pltpu
from jax.experimental.pallas import tpu_sc as plsc

N = 10000
E = 320000
D = 128
H = 128
C = 47

NC = 2
NS = 16
HD = D // 2
CHUNK = 128
NCHUNK = 160
EPW = CHUNK * NCHUNK
EPAD = EPW * NS
NACC = 10112
RPS = NACC // NS


NPHASE = 8
HCHUNK = NCHUNK // NPHASE


def _propagate_body(z, srcp, dstp, zzero, out, acc, zsp, srcv, dstv,
                    rows, gsems, ssems):
    c = lax.axis_index("c")
    s = lax.axis_index("s")
    pltpu.sync_copy(zzero.at[pl.ds(s * RPS, RPS)], acc.at[pl.ds(s * RPS, RPS)])
    pltpu.sync_copy(z.at[c, pl.ds(s * RPS, RPS)], zsp.at[pl.ds(s * RPS, RPS)])

    def issue_g(b, j):
        pltpu.async_copy(zsp.at[srcv.at[j]], rows[b], gsems[b])

    def wait_g(b):
        pltpu.make_async_copy(zsp.at[srcv.at[0]], rows[b], gsems[b]).wait()

    def issue_s(b, j):
        pltpu.async_copy(rows[b], acc.at[dstv.at[j]], ssems[b], add=True)

    def wait_s(b):
        pltpu.make_async_copy(rows[b], acc.at[dstv.at[0]], ssems[b]).wait()

    for phase in range(NPHASE):
        pltpu.sync_copy(srcp.at[s, pl.ds(phase * HCHUNK, HCHUNK)], srcv)
        pltpu.sync_copy(dstp.at[s, pl.ds(phase * HCHUNK, HCHUNK)], dstv)
        if phase == 0:
            plsc.subcore_barrier()
        issue_g(0, 0)
        issue_g(1, 1)
        wait_g(0)
        issue_s(0, 0)
        issue_g(2, 2)
        wait_g(1)
        issue_s(1, 1)
        issue_g(3, 3)

        def steady(j2, carry):
            j0 = 2 + 4 * j2
            for b4 in range(4):
                j = j0 + b4
                b = (2 + b4) % 4
                cc = b4
                wait_g(b)
                issue_s(b, j)
                wait_s(cc)
                issue_g(cc, j + 2)
            return carry

        lax.fori_loop(0, (HCHUNK - 4) // 4, steady, 0)
        wait_g(2)
        issue_s(2, HCHUNK - 2)
        wait_s(0)
        wait_g(3)
        issue_s(3, HCHUNK - 1)
        wait_s(1)
        wait_s(2)
        wait_s(3)

    plsc.subcore_barrier()
    pltpu.sync_copy(acc.at[pl.ds(s * RPS, RPS)], out.at[c, pl.ds(s * RPS, RPS)])


@functools.partial(
    pl.kernel,
    out_type=jax.ShapeDtypeStruct((NC, NACC, HD), jnp.float32),
    mesh=plsc.VectorSubcoreMesh(core_axis_name="c", subcore_axis_name="s"),
    scratch_types=[
        pltpu.VMEM_SHARED((NACC, HD), jnp.float32),
        pltpu.VMEM_SHARED((NACC, HD), jnp.float32),
        pltpu.VMEM((HCHUNK, CHUNK), jnp.int32),
        pltpu.VMEM((HCHUNK, CHUNK), jnp.int32),
        [pltpu.VMEM((CHUNK, HD), jnp.float32) for _ in range(4)],
        [pltpu.SemaphoreType.DMA for _ in range(4)],
        [pltpu.SemaphoreType.DMA for _ in range(4)],
    ],
    compiler_params=pltpu.CompilerParams(use_tc_tiling_on_sc=False),
)
def _propagate(z, srcp, dstp, zzero, out, acc, zsp, srcv, dstv, rows, gsems, ssems):
    _propagate_body(z, srcp, dstp, zzero, out, acc, zsp, srcv, dstv, rows, gsems, ssems)


def _split_cols(o_ref, res):
    o_ref[0, :N, :] = res[:, :HD]
    o_ref[1, :N, :] = res[:, HD:]
    pad = jnp.zeros((NACC - N, HD), jnp.float32)
    o_ref[0, N:, :] = pad
    o_ref[1, N:, :] = pad


def _mm_first_body(x_ref, w_ref, o_ref):
    res = lax.dot_general(
        x_ref[...], w_ref[...], (((1,), (1,)), ((), ())),
        preferred_element_type=jnp.float32)
    _split_cols(o_ref, res)


def _fuse_body(p_ref, b_ref, w_ref, o_ref):
    h = jnp.concatenate([p_ref[0, :N, :], p_ref[1, :N, :]], axis=1) + b_ref[...]
    h = jnp.maximum(h, 0.0)
    res = lax.dot_general(
        h, w_ref[...], (((1,), (1,)), ((), ())),
        preferred_element_type=jnp.float32)
    _split_cols(o_ref, res)


def _final_body(p_ref, b_ref, o_ref):
    v = jnp.concatenate([p_ref[0, :N, :], p_ref[1, :N, :]], axis=1) + b_ref[...]
    col = lax.broadcasted_iota(jnp.int32, (N, D), 1)
    valid = col < C
    vm = jnp.where(valid, v, -jnp.inf)
    m = jnp.max(vm, axis=1, keepdims=True)
    ex = jnp.where(valid, jnp.exp(v - m), 0.0)
    lse = jnp.log(jnp.sum(ex, axis=1, keepdims=True)) + m
    o_ref[...] = v - lse


_mm_first = pl.pallas_call(
    _mm_first_body,
    out_shape=jax.ShapeDtypeStruct((NC, NACC, HD), jnp.float32),
)

_fuse = pl.pallas_call(
    _fuse_body,
    out_shape=jax.ShapeDtypeStruct((NC, NACC, HD), jnp.float32),
)

_final = pl.pallas_call(
    _final_body,
    out_shape=jax.ShapeDtypeStruct((N, D), jnp.float32),
)


def kernel(x, edge_index, W1, b1, W2, b2, W3, b3):
    src = edge_index[0].astype(jnp.int32)
    dst = edge_index[1].astype(jnp.int32)
    pad = EPAD - E
    srcp = jnp.concatenate([src, jnp.zeros((pad,), jnp.int32)]).reshape(NS, NCHUNK, CHUNK)
    dstp = jnp.concatenate([dst, jnp.full((pad,), N, jnp.int32)]).reshape(NS, NCHUNK, CHUNK)
    zzero = jnp.zeros((NACC, HD), jnp.float32)

    W3p = jnp.zeros((D, H), jnp.float32).at[:C, :].set(W3)
    b3p = jnp.zeros((1, D), jnp.float32).at[0, :C].set(b3)

    z1 = _mm_first(x, W1)
    p1 = _propagate(z1, srcp, dstp, zzero)
    z2 = _fuse(p1, b1.reshape(1, H), W2)
    p2 = _propagate(z2, srcp, dstp, zzero)
    z3 = _fuse(p2, b2.reshape(1, H), W3p)
    p3 = _propagate(z3, srcp, dstp, zzero)
    o = _final(p3, b3p)
    return o[:, :C]


# speedup vs baseline: 10.4781x; 1.1176x over previous
import functools

import jax
import jax.numpy as jnp
from jax import lax
from jax.experimental import pallas as pl
from jax.experimental.pallas import tpu as pltpu
from jax.experimental.pallas import tpu_sc as plsc

N = 10000
E = 320000
D = 128
H = 128
C = 47

NC = 2
NS = 16
HD = D // 2
CP = 64
HD3 = CP // 2
CHUNK = 128
NCHUNK = 160
EPW = CHUNK * NCHUNK
EPAD = EPW * NS
NACC = 10112
RPS = NACC // NS


NPHASE = 8
HCHUNK = NCHUNK // NPHASE


def _propagate_body(z, srcp, dstp, zzero, out, acc, zsp, srcv, dstv,
                    rows, gsems, ssems):
    c = lax.axis_index("c")
    s = lax.axis_index("s")
    pltpu.sync_copy(zzero.at[pl.ds(s * RPS, RPS)], acc.at[pl.ds(s * RPS, RPS)])
    pltpu.sync_copy(z.at[c, pl.ds(s * RPS, RPS)], zsp.at[pl.ds(s * RPS, RPS)])

    def issue_g(b, j):
        pltpu.async_copy(zsp.at[srcv.at[j]], rows[b], gsems[b])

    def wait_g(b):
        pltpu.make_async_copy(zsp.at[srcv.at[0]], rows[b], gsems[b]).wait()

    def issue_s(b, j):
        pltpu.async_copy(rows[b], acc.at[dstv.at[j]], ssems[b], add=True)

    def wait_s(b):
        pltpu.make_async_copy(rows[b], acc.at[dstv.at[0]], ssems[b]).wait()

    for phase in range(NPHASE):
        pltpu.sync_copy(srcp.at[s, pl.ds(phase * HCHUNK, HCHUNK)], srcv)
        pltpu.sync_copy(dstp.at[s, pl.ds(phase * HCHUNK, HCHUNK)], dstv)
        if phase == 0:
            plsc.subcore_barrier()
        issue_g(0, 0)
        issue_g(1, 1)
        wait_g(0)
        issue_s(0, 0)
        issue_g(2, 2)
        wait_g(1)
        issue_s(1, 1)
        issue_g(3, 3)

        def steady(j2, carry):
            j0 = 2 + 4 * j2
            for b4 in range(4):
                j = j0 + b4
                b = (2 + b4) % 4
                cc = b4
                wait_g(b)
                issue_s(b, j)
                wait_s(cc)
                issue_g(cc, j + 2)
            return carry

        lax.fori_loop(0, (HCHUNK - 4) // 4, steady, 0)
        wait_g(2)
        issue_s(2, HCHUNK - 2)
        wait_s(0)
        wait_g(3)
        issue_s(3, HCHUNK - 1)
        wait_s(1)
        wait_s(2)
        wait_s(3)

    plsc.subcore_barrier()
    pltpu.sync_copy(acc.at[pl.ds(s * RPS, RPS)], out.at[c, pl.ds(s * RPS, RPS)])


def _make_propagate(hd):
    @functools.partial(
        pl.kernel,
        out_type=jax.ShapeDtypeStruct((NC, NACC, hd), jnp.float32),
        mesh=plsc.VectorSubcoreMesh(core_axis_name="c", subcore_axis_name="s"),
        scratch_types=[
            pltpu.VMEM_SHARED((NACC, hd), jnp.float32),
            pltpu.VMEM_SHARED((NACC, hd), jnp.float32),
            pltpu.VMEM((HCHUNK, CHUNK), jnp.int32),
            pltpu.VMEM((HCHUNK, CHUNK), jnp.int32),
            [pltpu.VMEM((CHUNK, hd), jnp.float32) for _ in range(4)],
            [pltpu.SemaphoreType.DMA for _ in range(4)],
            [pltpu.SemaphoreType.DMA for _ in range(4)],
        ],
        compiler_params=pltpu.CompilerParams(use_tc_tiling_on_sc=False),
    )
    def prop(z, srcp, dstp, zzero, out, acc, zsp, srcv, dstv, rows, gsems, ssems):
        _propagate_body(z, srcp, dstp, zzero, out, acc, zsp, srcv, dstv,
                        rows, gsems, ssems)

    return prop


_propagate = _make_propagate(HD)
_propagate3 = _make_propagate(HD3)


def _split_cols(o_ref, res):
    hd = res.shape[1] // 2
    o_ref[0, :N, :] = res[:, :hd]
    o_ref[1, :N, :] = res[:, hd:]
    pad = jnp.zeros((NACC - N, hd), jnp.float32)
    o_ref[0, N:, :] = pad
    o_ref[1, N:, :] = pad


def _mm_first_body(x_ref, w_ref, o_ref):
    res = lax.dot_general(
        x_ref[...], w_ref[...], (((1,), (1,)), ((), ())),
        preferred_element_type=jnp.float32)
    _split_cols(o_ref, res)


def _fuse_body(p_ref, b_ref, w_ref, o_ref):
    h = jnp.concatenate([p_ref[0, :N, :], p_ref[1, :N, :]], axis=1) + b_ref[...]
    h = jnp.maximum(h, 0.0)
    res = lax.dot_general(
        h, w_ref[...], (((1,), (1,)), ((), ())),
        preferred_element_type=jnp.float32)
    _split_cols(o_ref, res)


def _final_body(p_ref, b_ref, o_ref):
    v = jnp.concatenate([p_ref[0, :N, :], p_ref[1, :N, :]], axis=1) + b_ref[...]
    col = lax.broadcasted_iota(jnp.int32, (N, CP), 1)
    valid = col < C
    vm = jnp.where(valid, v, -jnp.inf)
    m = jnp.max(vm, axis=1, keepdims=True)
    ex = jnp.where(valid, jnp.exp(v - m), 0.0)
    lse = jnp.log(jnp.sum(ex, axis=1, keepdims=True)) + m
    o_ref[...] = v - lse


_mm_first = pl.pallas_call(
    _mm_first_body,
    out_shape=jax.ShapeDtypeStruct((NC, NACC, HD), jnp.float32),
)

_fuse = pl.pallas_call(
    _fuse_body,
    out_shape=jax.ShapeDtypeStruct((NC, NACC, HD), jnp.float32),
)

_fuse3 = pl.pallas_call(
    _fuse_body,
    out_shape=jax.ShapeDtypeStruct((NC, NACC, HD3), jnp.float32),
)

_final = pl.pallas_call(
    _final_body,
    out_shape=jax.ShapeDtypeStruct((N, CP), jnp.float32),
)


def kernel(x, edge_index, W1, b1, W2, b2, W3, b3):
    src = edge_index[0].astype(jnp.int32)
    dst = edge_index[1].astype(jnp.int32)
    pad = EPAD - E
    srcp = jnp.concatenate([src, jnp.zeros((pad,), jnp.int32)]).reshape(NS, NCHUNK, CHUNK)
    dstp = jnp.concatenate([dst, jnp.full((pad,), N, jnp.int32)]).reshape(NS, NCHUNK, CHUNK)
    zzero = jnp.zeros((NACC, HD), jnp.float32)
    zzero3 = jnp.zeros((NACC, HD3), jnp.float32)

    W3p = jnp.zeros((CP, H), jnp.float32).at[:C, :].set(W3)
    b3p = jnp.zeros((1, CP), jnp.float32).at[0, :C].set(b3)

    z1 = _mm_first(x, W1)
    p1 = _propagate(z1, srcp, dstp, zzero)
    z2 = _fuse(p1, b1.reshape(1, H), W2)
    p2 = _propagate(z2, srcp, dstp, zzero)
    z3 = _fuse3(p2, b2.reshape(1, H), W3p)
    p3 = _propagate3(z3, srcp, dstp, zzero3)
    o = _final(p3, b3p)
    return o[:, :C]


# speedup vs baseline: 11.0635x; 1.0559x over previous
import functools

import jax
import jax.numpy as jnp
from jax import lax
from jax.experimental import pallas as pl
from jax.experimental.pallas import tpu as pltpu
from jax.experimental.pallas import tpu_sc as plsc

N = 10000
E = 320000
D = 128
H = 128
C = 47

NC = 2
NS = 16
HD = D // 2
CP = 64
HD3 = CP // 2
CHUNK = 128
NCHUNK = 160
EPW = CHUNK * NCHUNK
EPAD = EPW * NS
NACC = 10112
RPS = NACC // NS


NPHASE = 4
HCHUNK = NCHUNK // NPHASE


def _propagate_body(z, srcp, dstp, zzero, out, acc, zsp, srcv, dstv,
                    rows, gsems, ssems):
    c = lax.axis_index("c")
    s = lax.axis_index("s")
    pltpu.sync_copy(zzero.at[pl.ds(s * RPS, RPS)], acc.at[pl.ds(s * RPS, RPS)])
    pltpu.sync_copy(z.at[c, pl.ds(s * RPS, RPS)], zsp.at[pl.ds(s * RPS, RPS)])

    def issue_g(b, j):
        pltpu.async_copy(zsp.at[srcv.at[j]], rows[b], gsems[b])

    def wait_g(b):
        pltpu.make_async_copy(zsp.at[srcv.at[0]], rows[b], gsems[b]).wait()

    def issue_s(b, j):
        pltpu.async_copy(rows[b], acc.at[dstv.at[j]], ssems[b], add=True)

    def wait_s(b):
        pltpu.make_async_copy(rows[b], acc.at[dstv.at[0]], ssems[b]).wait()

    for phase in range(NPHASE):
        pltpu.sync_copy(srcp.at[s, pl.ds(phase * HCHUNK, HCHUNK)], srcv)
        pltpu.sync_copy(dstp.at[s, pl.ds(phase * HCHUNK, HCHUNK)], dstv)
        if phase == 0:
            plsc.subcore_barrier()
        issue_g(0, 0)
        issue_g(1, 1)
        wait_g(0)
        issue_s(0, 0)
        issue_g(2, 2)
        wait_g(1)
        issue_s(1, 1)
        issue_g(3, 3)

        def steady(j2, carry):
            j0 = 2 + 4 * j2
            for b4 in range(4):
                j = j0 + b4
                b = (2 + b4) % 4
                cc = b4
                wait_g(b)
                issue_s(b, j)
                wait_s(cc)
                issue_g(cc, j + 2)
            return carry

        lax.fori_loop(0, (HCHUNK - 4) // 4, steady, 0)
        wait_g(2)
        issue_s(2, HCHUNK - 2)
        wait_s(0)
        wait_g(3)
        issue_s(3, HCHUNK - 1)
        wait_s(1)
        wait_s(2)
        wait_s(3)

    plsc.subcore_barrier()
    pltpu.sync_copy(acc.at[pl.ds(s * RPS, RPS)], out.at[c, pl.ds(s * RPS, RPS)])


def _make_propagate(hd):
    @functools.partial(
        pl.kernel,
        out_type=jax.ShapeDtypeStruct((NC, NACC, hd), jnp.float32),
        mesh=plsc.VectorSubcoreMesh(core_axis_name="c", subcore_axis_name="s"),
        scratch_types=[
            pltpu.VMEM_SHARED((NACC, hd), jnp.float32),
            pltpu.VMEM_SHARED((NACC, hd), jnp.float32),
            pltpu.VMEM((HCHUNK, CHUNK), jnp.int32),
            pltpu.VMEM((HCHUNK, CHUNK), jnp.int32),
            [pltpu.VMEM((CHUNK, hd), jnp.float32) for _ in range(4)],
            [pltpu.SemaphoreType.DMA for _ in range(4)],
            [pltpu.SemaphoreType.DMA for _ in range(4)],
        ],
        compiler_params=pltpu.CompilerParams(use_tc_tiling_on_sc=False),
    )
    def prop(z, srcp, dstp, zzero, out, acc, zsp, srcv, dstv, rows, gsems, ssems):
        _propagate_body(z, srcp, dstp, zzero, out, acc, zsp, srcv, dstv,
                        rows, gsems, ssems)

    return prop


_propagate = _make_propagate(HD)
_propagate3 = _make_propagate(HD3)


def _split_cols(o_ref, res):
    hd = res.shape[1] // 2
    o_ref[0, :N, :] = res[:, :hd]
    o_ref[1, :N, :] = res[:, hd:]
    pad = jnp.zeros((NACC - N, hd), jnp.float32)
    o_ref[0, N:, :] = pad
    o_ref[1, N:, :] = pad


def _mm_first_body(x_ref, w_ref, o_ref):
    res = lax.dot_general(
        x_ref[...], w_ref[...], (((1,), (1,)), ((), ())),
        preferred_element_type=jnp.float32)
    _split_cols(o_ref, res)


def _fuse_body(p_ref, b_ref, w_ref, o_ref):
    h = jnp.concatenate([p_ref[0, :N, :], p_ref[1, :N, :]], axis=1) + b_ref[...]
    h = jnp.maximum(h, 0.0)
    res = lax.dot_general(
        h, w_ref[...], (((1,), (1,)), ((), ())),
        preferred_element_type=jnp.float32)
    _split_cols(o_ref, res)


def _final_body(p_ref, b_ref, o_ref):
    v = jnp.concatenate([p_ref[0, :N, :], p_ref[1, :N, :]], axis=1) + b_ref[...]
    col = lax.broadcasted_iota(jnp.int32, (N, CP), 1)
    valid = col < C
    vm = jnp.where(valid, v, -jnp.inf)
    m = jnp.max(vm, axis=1, keepdims=True)
    ex = jnp.where(valid, jnp.exp(v - m), 0.0)
    lse = jnp.log(jnp.sum(ex, axis=1, keepdims=True)) + m
    o_ref[...] = v - lse


_mm_first = pl.pallas_call(
    _mm_first_body,
    out_shape=jax.ShapeDtypeStruct((NC, NACC, HD), jnp.float32),
)

_fuse = pl.pallas_call(
    _fuse_body,
    out_shape=jax.ShapeDtypeStruct((NC, NACC, HD), jnp.float32),
)

_fuse3 = pl.pallas_call(
    _fuse_body,
    out_shape=jax.ShapeDtypeStruct((NC, NACC, HD3), jnp.float32),
)

_final = pl.pallas_call(
    _final_body,
    out_shape=jax.ShapeDtypeStruct((N, CP), jnp.float32),
)


def kernel(x, edge_index, W1, b1, W2, b2, W3, b3):
    src = edge_index[0].astype(jnp.int32)
    dst = edge_index[1].astype(jnp.int32)
    pad = EPAD - E
    srcp = jnp.concatenate([src, jnp.zeros((pad,), jnp.int32)]).reshape(NS, NCHUNK, CHUNK)
    dstp = jnp.concatenate([dst, jnp.full((pad,), N, jnp.int32)]).reshape(NS, NCHUNK, CHUNK)
    zzero = jnp.zeros((NACC, HD), jnp.float32)
    zzero3 = jnp.zeros((NACC, HD3), jnp.float32)

    W3p = jnp.zeros((CP, H), jnp.float32).at[:C, :].set(W3)
    b3p = jnp.zeros((1, CP), jnp.float32).at[0, :C].set(b3)

    z1 = _mm_first(x, W1)
    p1 = _propagate(z1, srcp, dstp, zzero)
    z2 = _fuse(p1, b1.reshape(1, H), W2)
    p2 = _propagate(z2, srcp, dstp, zzero)
    z3 = _fuse3(p2, b2.reshape(1, H), W3p)
    p3 = _propagate3(z3, srcp, dstp, zzero3)
    o = _final(p3, b3p)
    return o[:, :C]
